# baseline (speedup 1.0000x reference)
import functools

import jax
import jax.numpy as jnp
from jax.experimental import pallas as pl

N = 1_000_000
ELE_DIM = 32
MAT_FEAT = 83
TILE = 8_000


def _decoder_body(x_ref, wt_ref, b_ref, out_ref, mask_ref):
    x = x_ref[...]
    mask = jnp.any(x != 0.0, axis=1)
    y = jnp.dot(x, wt_ref[...], preferred_element_type=jnp.float32)
    y = y + b_ref[...]
    y = y * 0.5 * (1.0 + jax.lax.erf(y * 0.7071067811865476))
    y = jnp.where(mask[:, None], y, 0.0)
    out_ref[...] = jax.nn.sigmoid(y)
    mask_ref[...] = mask[None, None, :]


@functools.partial(jax.jit, static_argnames=("interpret",))
def _decoder(inputs, wt, b2, interpret=False):
    n = inputs.shape[0]
    grid = (n // TILE,)
    out, mask = pl.pallas_call(
        _decoder_body,
        grid=grid,
        in_specs=[
            pl.BlockSpec((TILE, ELE_DIM), lambda i: (i, 0)),
            pl.BlockSpec((ELE_DIM, MAT_FEAT), lambda i: (0, 0)),
            pl.BlockSpec((1, MAT_FEAT), lambda i: (0, 0)),
        ],
        out_specs=[
            pl.BlockSpec((TILE, MAT_FEAT), lambda i: (i, 0)),
            pl.BlockSpec((1, 1, TILE), lambda i: (i, 0, 0)),
        ],
        out_shape=[
            jax.ShapeDtypeStruct((n, MAT_FEAT), jnp.float32),
            jax.ShapeDtypeStruct((n // TILE, 1, TILE), jnp.bool_),
        ],
        interpret=interpret,
    )(inputs, wt, b2)
    return out, mask.reshape(n)


def kernel(inputs, W, b):
    wt = W.T
    b2 = b.reshape(1, MAT_FEAT)
    return _decoder(inputs, wt, b2)


# speedup vs baseline: 1.0001x; 1.0001x over previous
import functools

import jax
import jax.numpy as jnp
from jax.experimental import pallas as pl

N = 1_000_000
ELE_DIM = 32
MAT_FEAT = 83
TILE = 20_000


def _decoder_body(x_ref, wt_ref, b_ref, out_ref, mask_ref):
    x = x_ref[...]
    mask = jnp.any(x != 0.0, axis=1)
    y = jnp.dot(x, wt_ref[...], preferred_element_type=jnp.float32)
    y = y + b_ref[...]
    y = y * 0.5 * (1.0 + jax.lax.erf(y * 0.7071067811865476))
    y = jnp.where(mask[:, None], y, 0.0)
    out_ref[...] = jax.nn.sigmoid(y)
    mask_ref[...] = mask[None, None, :]


@functools.partial(jax.jit, static_argnames=("interpret",))
def _decoder(inputs, wt, b2, interpret=False):
    n = inputs.shape[0]
    grid = (n // TILE,)
    out, mask = pl.pallas_call(
        _decoder_body,
        grid=grid,
        in_specs=[
            pl.BlockSpec((TILE, ELE_DIM), lambda i: (i, 0)),
            pl.BlockSpec((ELE_DIM, MAT_FEAT), lambda i: (0, 0)),
            pl.BlockSpec((1, MAT_FEAT), lambda i: (0, 0)),
        ],
        out_specs=[
            pl.BlockSpec((TILE, MAT_FEAT), lambda i: (i, 0)),
            pl.BlockSpec((1, 1, TILE), lambda i: (i, 0, 0)),
        ],
        out_shape=[
            jax.ShapeDtypeStruct((n, MAT_FEAT), jnp.float32),
            jax.ShapeDtypeStruct((n // TILE, 1, TILE), jnp.bool_),
        ],
        interpret=interpret,
    )(inputs, wt, b2)
    return out, mask.reshape(n)


def kernel(inputs, W, b):
    wt = W.T
    b2 = b.reshape(1, MAT_FEAT)
    return _decoder(inputs, wt, b2)


# speedup vs baseline: 1.6521x; 1.6519x over previous
import functools

import jax
import jax.numpy as jnp
from jax.experimental import pallas as pl

N = 1_000_000
ELE_DIM = 32
MAT_FEAT = 83
TILE = 8_192


def _decoder_body(x_ref, w_ref, b_ref, out_ref, mask_ref):
    x = x_ref[...]
    mask = jnp.any(x != 0.0, axis=0)
    y = jnp.dot(w_ref[...], x, preferred_element_type=jnp.float32)
    y = y + b_ref[...]
    y = y * 0.5 * (1.0 + jax.lax.erf(y * 0.7071067811865476))
    y = jnp.where(mask[None, :], y, 0.0)
    out_ref[...] = jax.nn.sigmoid(y)
    mask_ref[...] = mask


@functools.partial(jax.jit, static_argnames=("interpret",))
def _decoder(xt, w, b2, interpret=False):
    n = xt.shape[1]
    steps = pl.cdiv(n, TILE)
    out, mask = pl.pallas_call(
        _decoder_body,
        grid=(steps,),
        in_specs=[
            pl.BlockSpec((ELE_DIM, TILE), lambda i: (0, i)),
            pl.BlockSpec((MAT_FEAT, ELE_DIM), lambda i: (0, 0)),
            pl.BlockSpec((MAT_FEAT, 1), lambda i: (0, 0)),
        ],
        out_specs=[
            pl.BlockSpec((MAT_FEAT, TILE), lambda i: (0, i)),
            pl.BlockSpec((TILE,), lambda i: (i,)),
        ],
        out_shape=[
            jax.ShapeDtypeStruct((MAT_FEAT, n), jnp.float32),
            jax.ShapeDtypeStruct((steps * TILE,), jnp.bool_),
        ],
        interpret=interpret,
    )(xt, w, b2)
    return out, mask


def kernel(inputs, W, b):
    n = inputs.shape[0]
    b2 = b.reshape(MAT_FEAT, 1)
    out_t, mask = _decoder(inputs.T, W, b2)
    return out_t.T, mask[:n]


# speedup vs baseline: 5.5270x; 3.3455x over previous
import functools

import jax
import jax.numpy as jnp
from jax.experimental import pallas as pl

N = 1_000_000
ELE_DIM = 32
MAT_FEAT = 83
TILE = 8_192


def _decoder_body(x_ref, w_ref, b_ref, out_ref, mask_ref):
    x = x_ref[...]
    mask = jnp.any(x != 0.0, axis=0)
    y = jnp.dot(w_ref[...], x, preferred_element_type=jnp.float32)
    y = y + b_ref[...]
    y = y * 0.5 * (1.0 + jax.lax.erf(y * 0.7071067811865476))
    s = 1.0 / (1.0 + jnp.exp2(y * -1.4426950408889634))
    maskf = (mask[None, :]).astype(jnp.float32)
    out_ref[...] = 0.5 + maskf * (s - 0.5)
    mask_ref[...] = mask


@functools.partial(jax.jit, static_argnames=("interpret",))
def _decoder(xt, w, b2, interpret=False):
    n = xt.shape[1]
    steps = pl.cdiv(n, TILE)
    out, mask = pl.pallas_call(
        _decoder_body,
        grid=(steps,),
        in_specs=[
            pl.BlockSpec((ELE_DIM, TILE), lambda i: (0, i)),
            pl.BlockSpec((MAT_FEAT, ELE_DIM), lambda i: (0, 0)),
            pl.BlockSpec((MAT_FEAT, 1), lambda i: (0, 0)),
        ],
        out_specs=[
            pl.BlockSpec((MAT_FEAT, TILE), lambda i: (0, i)),
            pl.BlockSpec((TILE,), lambda i: (i,)),
        ],
        out_shape=[
            jax.ShapeDtypeStruct((MAT_FEAT, n), jnp.float32),
            jax.ShapeDtypeStruct((steps * TILE,), jnp.bool_),
        ],
        interpret=interpret,
    )(xt, w, b2)
    return out, mask


def kernel(inputs, W, b):
    n = inputs.shape[0]
    b2 = b.reshape(MAT_FEAT, 1)
    out_t, mask = _decoder(inputs.T, W, b2)
    return out_t.T, mask[:n]


# speedup vs baseline: 6.3265x; 1.1447x over previous
import functools

import jax
import jax.numpy as jnp
from jax.experimental import pallas as pl

N = 1_000_000
ELE_DIM = 32
MAT_FEAT = 83
TILE = 16_384


def _decoder_body(x_ref, w_ref, b_ref, out_ref, mask_ref):
    x = x_ref[...]
    mask = jnp.any(x != 0.0, axis=0)
    y = jnp.dot(w_ref[...], x, preferred_element_type=jnp.float32)
    y = y + b_ref[...]
    y = y * 0.5 * (1.0 + jax.lax.erf(y * 0.7071067811865476))
    s = 1.0 / (1.0 + jnp.exp2(y * -1.4426950408889634))
    maskf = (mask[None, :]).astype(jnp.float32)
    out_ref[...] = 0.5 + maskf * (s - 0.5)
    mask_ref[...] = mask


@functools.partial(jax.jit, static_argnames=("interpret",))
def _decoder(xt, w, b2, interpret=False):
    n = xt.shape[1]
    steps = pl.cdiv(n, TILE)
    out, mask = pl.pallas_call(
        _decoder_body,
        grid=(steps,),
        in_specs=[
            pl.BlockSpec((ELE_DIM, TILE), lambda i: (0, i)),
            pl.BlockSpec((MAT_FEAT, ELE_DIM), lambda i: (0, 0)),
            pl.BlockSpec((MAT_FEAT, 1), lambda i: (0, 0)),
        ],
        out_specs=[
            pl.BlockSpec((MAT_FEAT, TILE), lambda i: (0, i)),
            pl.BlockSpec((TILE,), lambda i: (i,)),
        ],
        out_shape=[
            jax.ShapeDtypeStruct((MAT_FEAT, n), jnp.float32),
            jax.ShapeDtypeStruct((steps * TILE,), jnp.bool_),
        ],
        interpret=interpret,
    )(xt, w, b2)
    return out, mask


def kernel(inputs, W, b):
    n = inputs.shape[0]
    b2 = b.reshape(MAT_FEAT, 1)
    out_t, mask = _decoder(inputs.T, W, b2)
    return out_t.T, mask[:n]


# speedup vs baseline: 6.4158x; 1.0141x over previous
import functools

import jax
import jax.numpy as jnp
from jax.experimental import pallas as pl

N = 1_000_000
ELE_DIM = 32
MAT_FEAT = 83
TILE = 32_768


def _decoder_body(x_ref, w_ref, b_ref, out_ref, mask_ref):
    x = x_ref[...]
    mask = jnp.any(x != 0.0, axis=0)
    y = jnp.dot(w_ref[...], x, preferred_element_type=jnp.float32)
    y = y + b_ref[...]
    y = y * 0.5 * (1.0 + jax.lax.erf(y * 0.7071067811865476))
    s = 1.0 / (1.0 + jnp.exp2(y * -1.4426950408889634))
    maskf = (mask[None, :]).astype(jnp.float32)
    out_ref[...] = 0.5 + maskf * (s - 0.5)
    mask_ref[...] = mask


@functools.partial(jax.jit, static_argnames=("interpret",))
def _decoder(xt, w, b2, interpret=False):
    n = xt.shape[1]
    steps = pl.cdiv(n, TILE)
    out, mask = pl.pallas_call(
        _decoder_body,
        grid=(steps,),
        in_specs=[
            pl.BlockSpec((ELE_DIM, TILE), lambda i: (0, i)),
            pl.BlockSpec((MAT_FEAT, ELE_DIM), lambda i: (0, 0)),
            pl.BlockSpec((MAT_FEAT, 1), lambda i: (0, 0)),
        ],
        out_specs=[
            pl.BlockSpec((MAT_FEAT, TILE), lambda i: (0, i)),
            pl.BlockSpec((TILE,), lambda i: (i,)),
        ],
        out_shape=[
            jax.ShapeDtypeStruct((MAT_FEAT, n), jnp.float32),
            jax.ShapeDtypeStruct((steps * TILE,), jnp.bool_),
        ],
        interpret=interpret,
    )(xt, w, b2)
    return out, mask


def kernel(inputs, W, b):
    n = inputs.shape[0]
    b2 = b.reshape(MAT_FEAT, 1)
    out_t, mask = _decoder(inputs.T, W, b2)
    return out_t.T, mask[:n]
